# baseline (speedup 1.0000x reference)
import functools
import numpy as np
import jax
import jax.numpy as jnp
from jax import lax
from jax.experimental import pallas as pl
from jax.experimental.pallas import tpu as pltpu
from jax.experimental.pallas import tpu_sc as plsc

_N = 2048
_K = (_N - 1) // 2
_INT_MIN = np.int32(-2147483648)


_ROWS_PER_BLOCK = 256


def _tc_median_block_kernel(x_ref, o_ref):
    x = x_ref[...]
    i = jax.lax.bitcast_convert_type(x, jnp.int32)
    int_min = jnp.int32(_INT_MIN)
    keys = jnp.where(i >= 0, i, int_min - i)

    res = jnp.zeros((x.shape[0], 1), dtype=jnp.int32)
    for b in range(31, -1, -1):
        bit = jnp.int32(np.array(1 << b, dtype=np.uint64).astype(np.uint32).view(np.int32))
        trial = res | bit
        trial_s = trial ^ int_min
        cnt = jnp.sum((keys < trial_s).astype(jnp.int32), axis=1, keepdims=True)
        res = jnp.where(cnt <= _K, trial, res)

    med_s = res ^ int_min
    med_i = jnp.where(med_s >= 0, med_s, int_min - med_s)
    o_ref[...] = jax.lax.bitcast_convert_type(med_i, jnp.float32)


def _tc_median(x2, row_start, nrows):
    blk0 = row_start // _ROWS_PER_BLOCK
    return pl.pallas_call(
        _tc_median_block_kernel,
        grid=(nrows // _ROWS_PER_BLOCK,),
        in_specs=[pl.BlockSpec((_ROWS_PER_BLOCK, _N), lambda g: (g + blk0, 0))],
        out_specs=pl.BlockSpec((_ROWS_PER_BLOCK, 1), lambda g: (g, 0)),
        out_shape=jax.ShapeDtypeStruct((nrows, 1), x2.dtype),
    )(x2)


_NW = 32
_GRP = 16


def _sc_median(x2, sc_rows):
    rows_per_w = sc_rows // _NW
    n_blocks = rows_per_w // _GRP
    mesh = plsc.VectorSubcoreMesh(core_axis_name="c", subcore_axis_name="s",
                                  num_cores=2, num_subcores=16)

    @functools.partial(
        pl.kernel,
        out_type=jax.ShapeDtypeStruct((sc_rows,), jnp.float32),
        mesh=mesh,
        scratch_types=[
            pltpu.VMEM((_GRP * _N,), jnp.float32),
            pltpu.VMEM((_GRP * _N,), jnp.int32),
            pltpu.VMEM((256 * 16,), jnp.int32),
            pltpu.VMEM((16 * 16,), jnp.int32),
            pltpu.VMEM((rows_per_w,), jnp.float32),
            pltpu.SemaphoreType.DMA,
        ],
        compiler_params=pltpu.CompilerParams(needs_layout_passes=False),
    )
    def sc_kernel(x_hbm, o_hbm, stage, work, hist8, hist4, outv, sem):
        int_min = jnp.int32(_INT_MIN)
        iota = lax.iota(jnp.int32, 16)
        ones = jnp.ones((16,), jnp.int32)
        zeros16 = jnp.zeros((16,), jnp.int32)
        wbase = iota * _N

        wid = lax.axis_index("s") * 2 + lax.axis_index("c")
        base = wid * rows_per_w

        def to_key(raw_f32):
            u = lax.bitcast_convert_type(raw_f32, jnp.int32)
            return u ^ ((u >> 31) | int_min)

        @plsc.parallel_loop(0, 256, unroll=8)
        def zero8(i):
            hist8[pl.ds(i * 16, 16)] = zeros16

        def blk_loop(blk, _):
            pltpu.sync_copy(
                x_hbm.at[pl.ds((base + blk * _GRP) * _N, _GRP * _N)], stage)

            @plsc.parallel_loop(0, _N, unroll=8)
            def hist_a(e):
                key = to_key(plsc.load_gather(stage, [wbase + e]))
                digit = lax.shift_right_logical(key, 24)
                plsc.addupdate_scatter(hist8, [digit * 16 + iota], ones)

            kk0 = jnp.full((16,), _K, jnp.int32)

            def walk_a(c, carry):
                cum, bin8, below, mnew, found = carry
                for bb in range(16):
                    b = c * 16 + bb
                    cnt = hist8[pl.ds(b * 16, 16)]
                    hist8[pl.ds(b * 16, 16)] = zeros16
                    ncum = cum + cnt
                    sel = jnp.logical_and(jnp.logical_not(found), ncum > kk0)
                    bin8 = jnp.where(sel, b, bin8)
                    below = jnp.where(sel, cum, below)
                    mnew = jnp.where(sel, cnt, mnew)
                    found = jnp.logical_or(found, sel)
                    cum = ncum
                return cum, bin8, below, mnew, found

            _, bin8, below, m, _ = lax.fori_loop(
                0, 16, walk_a,
                (zeros16, zeros16, zeros16, zeros16, iota < 0))
            kk = kk0 - below

            @plsc.parallel_loop(0, _N, unroll=8, carry=wbase)
            def compact_a(e, woff):
                key = to_key(plsc.load_gather(stage, [wbase + e]))
                digit = lax.shift_right_logical(key, 24)
                match = digit == bin8
                plsc.store_scatter(work, [woff], key, mask=match)
                return woff + match.astype(jnp.int32)

            def lvl_cond(carry):
                lv, kk, m = carry
                return jnp.logical_and(lv < 7, jnp.max(m) > 1)

            def lvl_body(carry):
                lv, kk, m = carry
                shift = 24 - 4 * lv
                maxm = jnp.max(m)
                for l in range(16):
                    hist4[pl.ds(l * 16, 16)] = zeros16

                @plsc.parallel_loop(0, maxm, unroll=4)
                def hist_l(e):
                    key = plsc.load_gather(work, [wbase + e])
                    digit = lax.shift_right_logical(key, shift) & 15
                    plsc.addupdate_scatter(hist4, [digit * 16 + iota], ones,
                                           mask=m > e)

                cum = zeros16
                bin4 = zeros16
                below = zeros16
                mnew = zeros16
                found = iota < 0
                for b in range(16):
                    cnt = hist4[pl.ds(b * 16, 16)]
                    ncum = cum + cnt
                    sel = jnp.logical_and(jnp.logical_not(found), ncum > kk)
                    bin4 = jnp.where(sel, b, bin4)
                    below = jnp.where(sel, cum, below)
                    mnew = jnp.where(sel, cnt, mnew)
                    found = jnp.logical_or(found, sel)
                    cum = ncum

                @plsc.parallel_loop(0, maxm, unroll=4, carry=wbase)
                def compact_l(e, woff):
                    key = plsc.load_gather(work, [wbase + e])
                    digit = lax.shift_right_logical(key, shift) & 15
                    match = jnp.logical_and(digit == bin4, m > e)
                    plsc.store_scatter(work, [woff], key, mask=match)
                    return woff + match.astype(jnp.int32)
                return lv + 1, kk - below, mnew

            lax.while_loop(lvl_cond, lvl_body, (1, kk, m))

            key = plsc.load_gather(work, [wbase])
            u = key ^ ((~(key >> 31)) | int_min)
            outv[pl.ds(blk * _GRP, 16)] = lax.bitcast_convert_type(u, jnp.float32)
            return 0

        lax.fori_loop(0, n_blocks, blk_loop, 0)
        pltpu.sync_copy(outv, o_hbm.at[pl.ds(base, rows_per_w)])

    return sc_kernel(x2.reshape(-1))


_SC_ROWS = 2560


def kernel(x):
    b, s, n = x.shape
    rows = b * s
    x2 = x.reshape(rows, n)
    out_sc = _sc_median(x2, _SC_ROWS)
    out_tc = _tc_median(x2, _SC_ROWS, rows - _SC_ROWS)
    out = jnp.concatenate([out_sc[:, None], out_tc], axis=0)
    return out.reshape(b, s, 1)


# speedup vs baseline: 1.2188x; 1.2188x over previous
import functools
import numpy as np
import jax
import jax.numpy as jnp
from jax import lax
from jax.experimental import pallas as pl
from jax.experimental.pallas import tpu as pltpu
from jax.experimental.pallas import tpu_sc as plsc

_N = 2048
_K = (_N - 1) // 2
_INT_MIN = np.int32(-2147483648)


_ROWS_PER_BLOCK = 256


def _tc_median_block_kernel(x_ref, o_ref):
    x = x_ref[...]
    i = jax.lax.bitcast_convert_type(x, jnp.int32)
    int_min = jnp.int32(_INT_MIN)
    keys = jnp.where(i >= 0, i, int_min - i)

    res = jnp.zeros((x.shape[0], 1), dtype=jnp.int32)
    for b in range(31, -1, -1):
        bit = jnp.int32(np.array(1 << b, dtype=np.uint64).astype(np.uint32).view(np.int32))
        trial = res | bit
        trial_s = trial ^ int_min
        cnt = jnp.sum((keys < trial_s).astype(jnp.int32), axis=1, keepdims=True)
        res = jnp.where(cnt <= _K, trial, res)

    med_s = res ^ int_min
    med_i = jnp.where(med_s >= 0, med_s, int_min - med_s)
    o_ref[...] = jax.lax.bitcast_convert_type(med_i, jnp.float32)


def _tc_median(x2, row_start, nrows):
    blk0 = row_start // _ROWS_PER_BLOCK
    return pl.pallas_call(
        _tc_median_block_kernel,
        grid=(nrows // _ROWS_PER_BLOCK,),
        in_specs=[pl.BlockSpec((_ROWS_PER_BLOCK, _N), lambda g: (g + blk0, 0))],
        out_specs=pl.BlockSpec((_ROWS_PER_BLOCK, 1), lambda g: (g, 0)),
        out_shape=jax.ShapeDtypeStruct((nrows, 1), x2.dtype),
    )(x2)


_NW = 32
_GRP = 16


def _sc_median(x2, sc_rows):
    rows_per_w = sc_rows // _NW
    n_blocks = rows_per_w // _GRP
    mesh = plsc.VectorSubcoreMesh(core_axis_name="c", subcore_axis_name="s",
                                  num_cores=2, num_subcores=16)

    @functools.partial(
        pl.kernel,
        out_type=jax.ShapeDtypeStruct((sc_rows,), jnp.float32),
        mesh=mesh,
        scratch_types=[
            pltpu.VMEM((_GRP, _N), jnp.float32),
            pltpu.VMEM((_GRP * _N,), jnp.int32),
            pltpu.VMEM((256 * 16,), jnp.int32),
            pltpu.VMEM((16 * 16,), jnp.int32),
            pltpu.VMEM((rows_per_w,), jnp.float32),
            pltpu.SemaphoreType.DMA,
        ],
        compiler_params=pltpu.CompilerParams(needs_layout_passes=False),
    )
    def sc_kernel(x_hbm, o_hbm, stage, work, hist8, hist4, outv, sem):
        int_min = jnp.int32(_INT_MIN)
        iota = lax.iota(jnp.int32, 16)
        ones = jnp.ones((16,), jnp.int32)
        zeros16 = jnp.zeros((16,), jnp.int32)
        wbase = iota * _N

        wid = lax.axis_index("s") * 2 + lax.axis_index("c")
        base = wid * rows_per_w

        def to_key(raw_f32):
            u = lax.bitcast_convert_type(raw_f32, jnp.int32)
            return u ^ ((u >> 31) | int_min)

        @plsc.parallel_loop(0, 256, unroll=8)
        def zero8(i):
            hist8[pl.ds(i * 16, 16)] = zeros16

        def blk_loop(blk, _):
            pltpu.sync_copy(x_hbm.at[pl.ds(base + blk * _GRP, _GRP)], stage)

            @plsc.parallel_loop(0, _N, unroll=8)
            def hist_a(e):
                key = to_key(plsc.load_gather(stage, [iota, e + 0 * iota]))
                digit = lax.shift_right_logical(key, 24)
                plsc.addupdate_scatter(hist8, [digit * 16 + iota], ones)

            kk0 = jnp.full((16,), _K, jnp.int32)

            def walk_a(c, carry):
                cum, bin8, below, mnew, found = carry
                for bb in range(16):
                    b = c * 16 + bb
                    cnt = hist8[pl.ds(b * 16, 16)]
                    hist8[pl.ds(b * 16, 16)] = zeros16
                    ncum = cum + cnt
                    sel = jnp.logical_and(jnp.logical_not(found), ncum > kk0)
                    bin8 = jnp.where(sel, b, bin8)
                    below = jnp.where(sel, cum, below)
                    mnew = jnp.where(sel, cnt, mnew)
                    found = jnp.logical_or(found, sel)
                    cum = ncum
                return cum, bin8, below, mnew, found

            _, bin8, below, m, _ = lax.fori_loop(
                0, 16, walk_a,
                (zeros16, zeros16, zeros16, zeros16, iota < 0))
            kk = kk0 - below

            @plsc.parallel_loop(0, _N, unroll=8, carry=wbase)
            def compact_a(e, woff):
                key = to_key(plsc.load_gather(stage, [iota, e + 0 * iota]))
                digit = lax.shift_right_logical(key, 24)
                match = digit == bin8
                plsc.store_scatter(work, [woff], key, mask=match)
                return woff + match.astype(jnp.int32)

            def lvl_cond(carry):
                lv, kk, m = carry
                return jnp.logical_and(lv < 7, jnp.max(m) > 1)

            def lvl_body(carry):
                lv, kk, m = carry
                shift = 24 - 4 * lv
                maxm = jnp.max(m)
                for l in range(16):
                    hist4[pl.ds(l * 16, 16)] = zeros16

                @plsc.parallel_loop(0, maxm, unroll=4)
                def hist_l(e):
                    key = plsc.load_gather(work, [wbase + e])
                    digit = lax.shift_right_logical(key, shift) & 15
                    plsc.addupdate_scatter(hist4, [digit * 16 + iota], ones,
                                           mask=m > e)

                cum = zeros16
                bin4 = zeros16
                below = zeros16
                mnew = zeros16
                found = iota < 0
                for b in range(16):
                    cnt = hist4[pl.ds(b * 16, 16)]
                    ncum = cum + cnt
                    sel = jnp.logical_and(jnp.logical_not(found), ncum > kk)
                    bin4 = jnp.where(sel, b, bin4)
                    below = jnp.where(sel, cum, below)
                    mnew = jnp.where(sel, cnt, mnew)
                    found = jnp.logical_or(found, sel)
                    cum = ncum

                @plsc.parallel_loop(0, maxm, unroll=4, carry=wbase)
                def compact_l(e, woff):
                    key = plsc.load_gather(work, [wbase + e])
                    digit = lax.shift_right_logical(key, shift) & 15
                    match = jnp.logical_and(digit == bin4, m > e)
                    plsc.store_scatter(work, [woff], key, mask=match)
                    return woff + match.astype(jnp.int32)
                return lv + 1, kk - below, mnew

            lax.while_loop(lvl_cond, lvl_body, (1, kk, m))

            key = plsc.load_gather(work, [wbase])
            u = key ^ ((~(key >> 31)) | int_min)
            outv[pl.ds(blk * _GRP, 16)] = lax.bitcast_convert_type(u, jnp.float32)
            return 0

        lax.fori_loop(0, n_blocks, blk_loop, 0)
        pltpu.sync_copy(outv, o_hbm.at[pl.ds(base, rows_per_w)])

    return sc_kernel(x2)


_SC_ROWS = 2560


def kernel(x):
    b, s, n = x.shape
    rows = b * s
    x2 = x.reshape(rows, n)
    out_sc = _sc_median(x2, _SC_ROWS)
    out_tc = _tc_median(x2, _SC_ROWS, rows - _SC_ROWS)
    out = jnp.concatenate([out_sc[:, None], out_tc], axis=0)
    return out.reshape(b, s, 1)


# speedup vs baseline: 1.2446x; 1.0212x over previous
import functools
import numpy as np
import jax
import jax.numpy as jnp
from jax import lax
from jax.experimental import pallas as pl
from jax.experimental.pallas import tpu as pltpu
from jax.experimental.pallas import tpu_sc as plsc

_N = 2048
_K = (_N - 1) // 2
_INT_MIN = np.int32(-2147483648)


_ROWS_PER_BLOCK = 256


def _tc_median_block_kernel(x_ref, o_ref):
    x = x_ref[...]
    i = jax.lax.bitcast_convert_type(x, jnp.int32)
    int_min = jnp.int32(_INT_MIN)
    keys = jnp.where(i >= 0, i, int_min - i)

    res = jnp.zeros((x.shape[0], 1), dtype=jnp.int32)
    for b in range(31, -1, -1):
        bit = jnp.int32(np.array(1 << b, dtype=np.uint64).astype(np.uint32).view(np.int32))
        trial = res | bit
        trial_s = trial ^ int_min
        cnt = jnp.sum((keys < trial_s).astype(jnp.int32), axis=1, keepdims=True)
        res = jnp.where(cnt <= _K, trial, res)

    med_s = res ^ int_min
    med_i = jnp.where(med_s >= 0, med_s, int_min - med_s)
    o_ref[...] = jax.lax.bitcast_convert_type(med_i, jnp.float32)


def _tc_median(x2, row_start, nrows):
    blk0 = row_start // _ROWS_PER_BLOCK
    return pl.pallas_call(
        _tc_median_block_kernel,
        grid=(nrows // _ROWS_PER_BLOCK,),
        in_specs=[pl.BlockSpec((_ROWS_PER_BLOCK, _N), lambda g: (g + blk0, 0))],
        out_specs=pl.BlockSpec((_ROWS_PER_BLOCK, 1), lambda g: (g, 0)),
        out_shape=jax.ShapeDtypeStruct((nrows, 1), x2.dtype),
    )(x2)


_NW = 32
_GRP = 16


def _sc_median(x2, sc_rows):
    rows_per_w = sc_rows // _NW
    n_blocks = rows_per_w // _GRP
    mesh = plsc.VectorSubcoreMesh(core_axis_name="c", subcore_axis_name="s",
                                  num_cores=2, num_subcores=16)

    @functools.partial(
        pl.kernel,
        out_type=jax.ShapeDtypeStruct((sc_rows,), jnp.float32),
        mesh=mesh,
        scratch_types=[
            pltpu.VMEM((_GRP * _N,), jnp.float32),
            pltpu.VMEM((_GRP * _N,), jnp.int32),
            pltpu.VMEM((256 * 16,), jnp.int32),
            pltpu.VMEM((16 * 16,), jnp.int32),
            pltpu.VMEM((rows_per_w,), jnp.float32),
            pltpu.SemaphoreType.DMA,
        ],
        compiler_params=pltpu.CompilerParams(needs_layout_passes=False),
    )
    def sc_kernel(x_hbm, o_hbm, stage, work, hist8, hist4, outv, sem):
        int_min = jnp.int32(_INT_MIN)
        iota = lax.iota(jnp.int32, 16)
        ones = jnp.ones((16,), jnp.int32)
        zeros16 = jnp.zeros((16,), jnp.int32)
        wbase = iota * _N

        wid = lax.axis_index("s") * 2 + lax.axis_index("c")
        base = wid * rows_per_w

        def to_key(raw_f32):
            u = lax.bitcast_convert_type(raw_f32, jnp.int32)
            return u ^ ((u >> 31) | int_min)

        @plsc.parallel_loop(0, 256, unroll=8)
        def zero8(i):
            hist8[pl.ds(i * 16, 16)] = zeros16

        def blk_loop(blk, _):
            descs = [
                pltpu.async_copy(x_hbm.at[base + blk * _GRP + j],
                                 stage.at[pl.ds(j * _N, _N)], sem)
                for j in range(_GRP)
            ]
            for d in descs:
                d.wait()

            @plsc.parallel_loop(0, _N, unroll=8, carry=wbase)
            def hist_a(e, pos):
                key = to_key(plsc.load_gather(stage, [pos]))
                digit = lax.shift_right_logical(key, 24)
                plsc.addupdate_scatter(hist8, [digit * 16 + iota], ones)
                return pos + 1

            kk0 = jnp.full((16,), _K, jnp.int32)

            def walk_a(c, carry):
                cum, bin8, below, mnew, found = carry
                for bb in range(16):
                    b = c * 16 + bb
                    cnt = hist8[pl.ds(b * 16, 16)]
                    hist8[pl.ds(b * 16, 16)] = zeros16
                    ncum = cum + cnt
                    sel = jnp.logical_and(jnp.logical_not(found), ncum > kk0)
                    bin8 = jnp.where(sel, b, bin8)
                    below = jnp.where(sel, cum, below)
                    mnew = jnp.where(sel, cnt, mnew)
                    found = jnp.logical_or(found, sel)
                    cum = ncum
                return cum, bin8, below, mnew, found

            _, bin8, below, m, _ = lax.fori_loop(
                0, 16, walk_a,
                (zeros16, zeros16, zeros16, zeros16, iota < 0))
            kk = kk0 - below

            @plsc.parallel_loop(0, _N, unroll=8, carry=(wbase, wbase))
            def compact_a(e, carry):
                woff, pos = carry
                key = to_key(plsc.load_gather(stage, [pos]))
                digit = lax.shift_right_logical(key, 24)
                match = digit == bin8
                plsc.store_scatter(work, [woff], key, mask=match)
                return woff + match.astype(jnp.int32), pos + 1

            def lvl_cond(carry):
                lv, kk, m = carry
                return jnp.logical_and(lv < 7, jnp.max(m) > 1)

            def lvl_body(carry):
                lv, kk, m = carry
                shift = 24 - 4 * lv
                maxm = jnp.max(m)
                for l in range(16):
                    hist4[pl.ds(l * 16, 16)] = zeros16

                @plsc.parallel_loop(0, maxm, unroll=4, carry=zeros16)
                def hist_l(e, ev):
                    key = plsc.load_gather(work, [wbase + ev])
                    digit = lax.shift_right_logical(key, shift) & 15
                    plsc.addupdate_scatter(hist4, [digit * 16 + iota], ones,
                                           mask=m > ev)
                    return ev + 1

                cum = zeros16
                bin4 = zeros16
                below = zeros16
                mnew = zeros16
                found = iota < 0
                for b in range(16):
                    cnt = hist4[pl.ds(b * 16, 16)]
                    ncum = cum + cnt
                    sel = jnp.logical_and(jnp.logical_not(found), ncum > kk)
                    bin4 = jnp.where(sel, b, bin4)
                    below = jnp.where(sel, cum, below)
                    mnew = jnp.where(sel, cnt, mnew)
                    found = jnp.logical_or(found, sel)
                    cum = ncum

                @plsc.parallel_loop(0, maxm, unroll=4, carry=(wbase, zeros16))
                def compact_l(e, carry):
                    woff, ev = carry
                    key = plsc.load_gather(work, [wbase + ev])
                    digit = lax.shift_right_logical(key, shift) & 15
                    match = jnp.logical_and(digit == bin4, m > ev)
                    plsc.store_scatter(work, [woff], key, mask=match)
                    return woff + match.astype(jnp.int32), ev + 1
                return lv + 1, kk - below, mnew

            lax.while_loop(lvl_cond, lvl_body, (1, kk, m))

            key = plsc.load_gather(work, [wbase])
            u = key ^ ((~(key >> 31)) | int_min)
            outv[pl.ds(blk * _GRP, 16)] = lax.bitcast_convert_type(u, jnp.float32)
            return 0

        lax.fori_loop(0, n_blocks, blk_loop, 0)
        pltpu.sync_copy(outv, o_hbm.at[pl.ds(base, rows_per_w)])

    return sc_kernel(x2)


_SC_ROWS = 2560


def kernel(x):
    b, s, n = x.shape
    rows = b * s
    x2 = x.reshape(rows, n)
    out_sc = _sc_median(x2, _SC_ROWS)
    out_tc = _tc_median(x2, _SC_ROWS, rows - _SC_ROWS)
    out = jnp.concatenate([out_sc[:, None], out_tc], axis=0)
    return out.reshape(b, s, 1)
